# baseline (speedup 1.0000x reference)
import jax
import jax.numpy as jnp
from jax import lax
from jax.experimental import pallas as pl
from jax.experimental.pallas import tpu as pltpu
from jax.experimental.pallas import tpu_sc as plsc

_NC = 1
_NS = 16
_NW = 13


def _sc_body(i_hbm, p_hbm, inds_hbm, out_hbm, indsv, flatv, flatb, rowsv,
             rowsb, pv, resv, sem, semb, semp):
    w = lax.axis_index("s") * _NC + lax.axis_index("c")

    @pl.when(w < _NW)
    def _():
        iota = lax.iota(jnp.int32, 16)

        @pl.when(w < _NW - 1)
        def _():
            cp_i = pltpu.async_copy(inds_hbm.at[pl.ds(w * 16, 16)],
                                    indsv.at[pl.ds(0, 16)], sem)
            cp_p0 = pltpu.async_copy(p_hbm, pv, semp)
            cp_i.wait()

        @pl.when(w == _NW - 1)
        def _():
            cp_i = pltpu.async_copy(inds_hbm.at[pl.ds(192, 8)],
                                    indsv.at[pl.ds(0, 8)], sem)
            cp_p1 = pltpu.async_copy(p_hbm, pv, semp)
            cp_i.wait()

        bound = jnp.where(w == _NW - 1, 6, 14)
        ie = jnp.minimum(iota * 2, bound)
        r = plsc.load_gather(indsv, [ie])
        c = plsc.load_gather(indsv, [ie + 1])
        flat = r * 28 + c
        flatv[...] = flat
        plsc.store_scatter(flatb, [iota - 4], flat,
                           mask=(iota >= 4) & (iota < 8))
        g0 = pltpu.async_copy(i_hbm.at[flatv.at[pl.ds(0, 4)]], rowsv, sem)
        g1 = pltpu.async_copy(i_hbm.at[flatb], rowsb, semb)
        g0.wait()
        pltpu.make_async_copy(p_hbm, pv, semp).wait()
        res = jnp.zeros((16,), jnp.float32)
        for j in range(4):
            acc = rowsv[j, pl.ds(0, 16)] * pv[pl.ds(0, 16)]
            for cb in range(1, 8):
                acc = acc + rowsv[j, pl.ds(cb * 16, 16)] * pv[pl.ds(cb * 16, 16)]
            res = jnp.where(iota == j, jnp.sum(acc), res)
        g1.wait()
        for j in range(4):
            acc = rowsb[j, pl.ds(0, 16)] * pv[pl.ds(0, 16)]
            for cb in range(1, 8):
                acc = acc + rowsb[j, pl.ds(cb * 16, 16)] * pv[pl.ds(cb * 16, 16)]
            res = jnp.where(iota == j + 4, jnp.sum(acc), res)
        resv[...] = res
        pltpu.sync_copy(resv.at[pl.ds(0, 8)], out_hbm.at[pl.ds(w * 8, 8)])


def kernel(I, p, inds):
    out = pl.kernel(
        _sc_body,
        out_type=jax.ShapeDtypeStruct((_NW * 8,), jnp.float32),
        mesh=plsc.VectorSubcoreMesh(
            core_axis_name="c", subcore_axis_name="s",
            num_cores=_NC, num_subcores=_NS),
        compiler_params=pltpu.CompilerParams(needs_layout_passes=False),
        scratch_types=[
            pltpu.VMEM((16,), jnp.int32),
            pltpu.VMEM((16,), jnp.int32),
            pltpu.VMEM((4,), jnp.int32),
            pltpu.VMEM((4, 128), jnp.float32),
            pltpu.VMEM((4, 128), jnp.float32),
            pltpu.VMEM((128,), jnp.float32),
            pltpu.VMEM((16,), jnp.float32),
            pltpu.SemaphoreType.DMA,
            pltpu.SemaphoreType.DMA,
            pltpu.SemaphoreType.DMA,
        ],
    )(I, p, inds)
    return jnp.reshape(out[:100], (50, 2))


# speedup vs baseline: 1.0013x; 1.0013x over previous
import jax
import jax.numpy as jnp
from jax import lax
from jax.experimental import pallas as pl
from jax.experimental.pallas import tpu as pltpu
from jax.experimental.pallas import tpu_sc as plsc

_NC = 1
_NS = 13
_NW = 13


def _sc_body(i_hbm, p_hbm, inds_hbm, out_hbm, indsv, flatv, rowsv, pv, resv,
             sem, semp):
    w = lax.axis_index("s") * _NC + lax.axis_index("c")

    @pl.when(w < _NW)
    def _():
        iota = lax.iota(jnp.int32, 16)

        @pl.when(w < _NW - 1)
        def _():
            cp_i = pltpu.async_copy(inds_hbm.at[pl.ds(w * 16, 16)],
                                    indsv.at[pl.ds(0, 16)], sem)
            cp_p0 = pltpu.async_copy(p_hbm, pv, semp)
            cp_i.wait()

        @pl.when(w == _NW - 1)
        def _():
            cp_i = pltpu.async_copy(inds_hbm.at[pl.ds(192, 8)],
                                    indsv.at[pl.ds(0, 8)], sem)
            cp_p1 = pltpu.async_copy(p_hbm, pv, semp)
            cp_i.wait()

        bound = jnp.where(w == _NW - 1, 6, 14)
        ie = jnp.minimum(iota * 2, bound)
        r = plsc.load_gather(indsv, [ie])
        c = plsc.load_gather(indsv, [ie + 1])
        flatv[...] = r * 28 + c
        pltpu.async_copy(i_hbm.at[flatv.at[pl.ds(0, 8)]], rowsv, sem).wait()
        pltpu.make_async_copy(p_hbm, pv, semp).wait()
        res = jnp.zeros((16,), jnp.float32)
        for j in range(8):
            acc = rowsv[j, pl.ds(0, 16)] * pv[pl.ds(0, 16)]
            for cb in range(1, 8):
                acc = acc + rowsv[j, pl.ds(cb * 16, 16)] * pv[pl.ds(cb * 16, 16)]
            res = jnp.where(iota == j, jnp.sum(acc), res)
        resv[...] = res
        pltpu.sync_copy(resv.at[pl.ds(0, 8)], out_hbm.at[pl.ds(w * 8, 8)])


def kernel(I, p, inds):
    out = pl.kernel(
        _sc_body,
        out_type=jax.ShapeDtypeStruct((_NW * 8,), jnp.float32),
        mesh=plsc.VectorSubcoreMesh(
            core_axis_name="c", subcore_axis_name="s",
            num_cores=_NC, num_subcores=_NS),
        compiler_params=pltpu.CompilerParams(needs_layout_passes=False),
        scratch_types=[
            pltpu.VMEM((16,), jnp.int32),
            pltpu.VMEM((16,), jnp.int32),
            pltpu.VMEM((8, 128), jnp.float32),
            pltpu.VMEM((128,), jnp.float32),
            pltpu.VMEM((16,), jnp.float32),
            pltpu.SemaphoreType.DMA,
            pltpu.SemaphoreType.DMA,
        ],
    )(I, p, inds)
    return jnp.reshape(out[:100], (50, 2))


# speedup vs baseline: 1.0072x; 1.0059x over previous
import jax
import jax.numpy as jnp
from jax import lax
from jax.experimental import pallas as pl
from jax.experimental.pallas import tpu as pltpu
from jax.experimental.pallas import tpu_sc as plsc

_NC = 1
_NS = 16
_NW = 13


def _sc_body(i_hbm, p_hbm, inds_hbm, out_hbm, indsv, flatv, rowsv, pv, resv,
             sem, semp):
    w = lax.axis_index("s") * _NC + lax.axis_index("c")

    @pl.when(w < _NW)
    def _():
        iota = lax.iota(jnp.int32, 16)

        @pl.when(w < _NW - 1)
        def _():
            cp_i = pltpu.async_copy(inds_hbm.at[pl.ds(w * 16, 16)],
                                    indsv.at[pl.ds(0, 16)], sem)
            pltpu.async_copy(p_hbm, pv, semp)
            cp_i.wait()

        @pl.when(w == _NW - 1)
        def _():
            cp_i = pltpu.async_copy(inds_hbm.at[pl.ds(192, 8)],
                                    indsv.at[pl.ds(0, 8)], sem)
            pltpu.async_copy(p_hbm, pv, semp)
            cp_i.wait()

        bound = jnp.where(w == _NW - 1, 6, 14)
        ie = jnp.minimum(iota * 2, bound)
        r = plsc.load_gather(indsv, [ie])
        c = plsc.load_gather(indsv, [ie + 1])
        flatv[...] = r * 28 + c
        pltpu.async_copy(i_hbm.at[flatv.at[pl.ds(0, 8)]], rowsv, sem).wait()
        pltpu.make_async_copy(p_hbm, pv, semp).wait()
        res = jnp.zeros((16,), jnp.float32)
        for j in range(8):
            acc = rowsv[j, pl.ds(0, 16)] * pv[pl.ds(0, 16)]
            for cb in range(1, 8):
                acc = acc + rowsv[j, pl.ds(cb * 16, 16)] * pv[pl.ds(cb * 16, 16)]
            res = jnp.where(iota == j, jnp.sum(acc), res)
        resv[...] = res
        pltpu.sync_copy(resv.at[pl.ds(0, 8)], out_hbm.at[pl.ds(w * 8, 8)])


def kernel(I, p, inds):
    out = pl.kernel(
        _sc_body,
        out_type=jax.ShapeDtypeStruct((_NW * 8,), jnp.float32),
        mesh=plsc.VectorSubcoreMesh(
            core_axis_name="c", subcore_axis_name="s",
            num_cores=_NC, num_subcores=_NS),
        compiler_params=pltpu.CompilerParams(needs_layout_passes=False),
        scratch_types=[
            pltpu.VMEM((16,), jnp.int32),
            pltpu.VMEM((16,), jnp.int32),
            pltpu.VMEM((8, 128), jnp.float32),
            pltpu.VMEM((128,), jnp.float32),
            pltpu.VMEM((16,), jnp.float32),
            pltpu.SemaphoreType.DMA,
            pltpu.SemaphoreType.DMA,
        ],
    )(I, p, inds)
    return jnp.reshape(out[:100], (50, 2))
